# baseline (speedup 1.0000x reference)
import functools

import jax
import jax.numpy as jnp
from jax import lax
from jax.experimental import pallas as pl
from jax.experimental.pallas import tpu as pltpu
from jax.experimental.pallas import tpu_sc as plsc

D = 256
F = 7
NW = 32
CHUNK = 32
L = 16
NT = 3
BASES = (0, 868, 2623)


def _merge_a(wa, wc, wd, wf, wh, war, o1, o2a, o3):
    o1[...] = wa[...][:, None, :] + war[...][None, :, :]
    o2a[...] = wc[...][:, None, :] + wh[...][None, :, :]
    o3[...] = wd[...][:, None, :] + wf[...][None, :, :]


def _merge_b(t_ch, wn, o2):
    o2[...] = t_ch[...][:, None, :] + wn[...][None, :, :]


def _build_merged_table(tables):
    wa, wc, wd, wf, wh, war, wn = tables
    o1, o2a, o3 = pl.pallas_call(
        _merge_a,
        out_shape=[
            jax.ShapeDtypeStruct((124, 7, D), jnp.float32),
            jax.ShapeDtypeStruct((9, 13, D), jnp.float32),
            jax.ShapeDtypeStruct((17, 22, D), jnp.float32),
        ],
    )(wa, wc, wd, wf, wh, war)
    o2 = pl.pallas_call(
        _merge_b,
        out_shape=jax.ShapeDtypeStruct((117, 15, D), jnp.float32),
    )(o2a.reshape(117, D), wn)
    return jnp.concatenate(
        [o1.reshape(868, D), o2.reshape(1755, D), o3.reshape(374, D)], axis=0
    )


def _make_sc_kernel(n, n_pad):
    bpw = n_pad // NW
    mesh = plsc.VectorSubcoreMesh(core_axis_name="c", subcore_axis_name="s")

    @functools.partial(
        pl.kernel,
        mesh=mesh,
        out_type=jax.ShapeDtypeStruct((n, D), jnp.float32),
        scratch_types=(
            [pltpu.VMEM((bpw,), jnp.int32) for _ in range(F)]
            + [pltpu.VMEM((NT * bpw,), jnp.int32)]
            + [pltpu.VMEM((NT * CHUNK, D), jnp.float32) for _ in range(2)]
            + [pltpu.SemaphoreType.DMA, pltpu.SemaphoreType.DMA]
        ),
    )
    def sc_kernel(table_hbm, idx_hbm, out_hbm, *scratch):
        raw = scratch[:F]
        cidx = scratch[F]
        rows = (scratch[F + 1], scratch[F + 2])
        sems = scratch[F + 3:]
        wid = lax.axis_index("s") * 2 + lax.axis_index("c")
        base = wid * bpw
        nc_w = jnp.maximum(jnp.minimum(n - base, bpw), 0) // CHUNK
        n_pairs = nc_w // 2

        for f in range(F):
            pltpu.sync_copy(idx_hbm.at[f, wid], raw[f])

        def combine(g, carry):
            s = pl.ds(g * L, L)
            o = (g // (CHUNK // L)) * (NT * CHUNK) + (g % (CHUNK // L)) * L
            cidx[pl.ds(o, L)] = raw[0][s] * 7 + raw[5][s]
            cidx[pl.ds(o + CHUNK, L)] = (
                (raw[1][s] * 13 + raw[4][s]) * 15 + raw[6][s] + BASES[1]
            )
            cidx[pl.ds(o + 2 * CHUNK, L)] = raw[2][s] * 22 + raw[3][s] + BASES[2]
            return carry

        lax.fori_loop(0, bpw // L, combine, 0)

        def issue(b, c):
            pltpu.async_copy(
                table_hbm.at[cidx.at[pl.ds(c * (NT * CHUNK), NT * CHUNK)]],
                rows[b], sems[b],
            )

        def drain(b, c):
            pltpu.make_async_copy(
                table_hbm.at[cidx.at[pl.ds(c * (NT * CHUNK), NT * CHUNK)]],
                rows[b], sems[b],
            ).wait()

        def acc_store(b, c):
            def acc_row(r, carry2):
                for k in range(D // L):
                    s = pl.ds(k * L, L)
                    rows[b][r, s] = (
                        rows[b][r, s]
                        + rows[b][CHUNK + r, s]
                        + rows[b][2 * CHUNK + r, s]
                    )
                return carry2

            lax.fori_loop(0, CHUNK, acc_row, 0)
            pltpu.sync_copy(rows[b].at[pl.ds(0, CHUNK)],
                            out_hbm.at[pl.ds(base + c * CHUNK, CHUNK)])

        issue(0, 0)

        def pair_body(i, carry):
            c0 = i * 2
            issue(1, c0 + 1)
            drain(0, c0)
            acc_store(0, c0)

            @pl.when(c0 + 2 < nc_w)
            def _():
                issue(0, c0 + 2)

            drain(1, c0 + 1)
            acc_store(1, c0 + 1)
            return carry

        lax.fori_loop(0, n_pairs, pair_body, 0)

        @pl.when(nc_w % 2 == 1)
        def _():
            drain(0, nc_w - 1)
            acc_store(0, nc_w - 1)

    return sc_kernel


def kernel(atomic_num, chiral_tag, degree, formal_charge, hybridization,
           is_aromatic, total_numHs, W_atomic_num, W_chiral_tag, W_degree,
           W_formal_charge, W_hybridization, W_is_aromatic, W_total_numHs):
    idxs = [atomic_num, chiral_tag, degree, formal_charge, hybridization,
            is_aromatic, total_numHs]
    tables = [W_atomic_num, W_chiral_tag, W_degree, W_formal_charge,
              W_hybridization, W_is_aromatic, W_total_numHs]
    n = atomic_num.shape[0]

    assert n % CHUNK == 0
    table = _build_merged_table(tables)
    grain = NW * L
    n_pad = ((n + grain - 1) // grain) * grain

    idx = jnp.stack([i.astype(jnp.int32) for i in idxs])
    idx = jnp.pad(idx, ((0, 0), (0, n_pad - n)))
    idx = idx.reshape(F, NW, n_pad // NW)

    return _make_sc_kernel(n, n_pad)(table, idx)


# speedup vs baseline: 1.0266x; 1.0266x over previous
import functools

import jax
import jax.numpy as jnp
from jax import lax
from jax.experimental import pallas as pl
from jax.experimental.pallas import tpu as pltpu
from jax.experimental.pallas import tpu_sc as plsc

D = 256
F = 7
NW = 32
CHUNK = 32
L = 16
NT = 3
BASES = (0, 868, 2623)


def _merge(wa, wc, wd, wf, wh, war, wn, o1, o2, o3):
    o1[...] = wa[...][:, None, :] + war[...][None, :, :]
    o2[...] = (
        wc[...][:, None, None, :]
        + wh[...][None, :, None, :]
        + wn[...][None, None, :, :]
    )
    o3[...] = wd[...][:, None, :] + wf[...][None, :, :]


def _build_merged_table(tables):
    wa, wc, wd, wf, wh, war, wn = tables
    o1, o2, o3 = pl.pallas_call(
        _merge,
        out_shape=[
            jax.ShapeDtypeStruct((124, 7, D), jnp.float32),
            jax.ShapeDtypeStruct((9, 13, 15, D), jnp.float32),
            jax.ShapeDtypeStruct((17, 22, D), jnp.float32),
        ],
    )(wa, wc, wd, wf, wh, war, wn)
    return jnp.concatenate(
        [o1.reshape(868, D), o2.reshape(1755, D), o3.reshape(374, D)], axis=0
    )


def _make_sc_kernel(n, n_pad):
    bpw = n_pad // NW
    mesh = plsc.VectorSubcoreMesh(core_axis_name="c", subcore_axis_name="s")

    @functools.partial(
        pl.kernel,
        mesh=mesh,
        out_type=jax.ShapeDtypeStruct((n, D), jnp.float32),
        scratch_types=(
            [pltpu.VMEM((bpw,), jnp.int32) for _ in range(F)]
            + [pltpu.VMEM((bpw,), jnp.int32) for _ in range(NT)]
            + [pltpu.VMEM((CHUNK, D), jnp.float32) for _ in range(2 * NT)]
            + [pltpu.SemaphoreType.DMA, pltpu.SemaphoreType.DMA]
        ),
    )
    def sc_kernel(table_hbm, idx_hbm, out_hbm, *scratch):
        raw = scratch[:F]
        cidx = scratch[F:F + NT]
        rows = (scratch[F + NT:F + 2 * NT], scratch[F + 2 * NT:F + 3 * NT])
        sems = scratch[F + 3 * NT:]
        wid = lax.axis_index("s") * 2 + lax.axis_index("c")
        base = wid * bpw
        nc_w = jnp.maximum(jnp.minimum(n - base, bpw), 0) // CHUNK
        n_pairs = nc_w // 2

        for f in range(F):
            pltpu.sync_copy(idx_hbm.at[f, wid], raw[f])

        def combine(g, carry):
            s = pl.ds(g * L, L)
            cidx[0][s] = raw[0][s] * 7 + raw[5][s]
            cidx[1][s] = (raw[1][s] * 13 + raw[4][s]) * 15 + raw[6][s] + BASES[1]
            cidx[2][s] = raw[2][s] * 22 + raw[3][s] + BASES[2]
            return carry

        lax.fori_loop(0, bpw // L, combine, 0)

        def issue(b, c):
            for t in range(NT):
                pltpu.async_copy(
                    table_hbm.at[cidx[t].at[pl.ds(c * CHUNK, CHUNK)]],
                    rows[b][t], sems[b],
                )

        def drain(b, c):
            for t in range(NT):
                pltpu.make_async_copy(
                    table_hbm.at[cidx[t].at[pl.ds(c * CHUNK, CHUNK)]],
                    rows[b][t], sems[b],
                ).wait()

        def acc_store(b, c):
            def acc_row(r, carry2):
                for k in range(D // L):
                    s = pl.ds(k * L, L)
                    rows[b][0][r, s] = (
                        rows[b][0][r, s] + rows[b][1][r, s] + rows[b][2][r, s]
                    )
                return carry2

            lax.fori_loop(0, CHUNK, acc_row, 0)
            pltpu.sync_copy(rows[b][0],
                            out_hbm.at[pl.ds(base + c * CHUNK, CHUNK)])

        issue(0, 0)

        def pair_body(i, carry):
            c0 = i * 2
            issue(1, c0 + 1)
            drain(0, c0)
            acc_store(0, c0)

            @pl.when(c0 + 2 < nc_w)
            def _():
                issue(0, c0 + 2)

            drain(1, c0 + 1)
            acc_store(1, c0 + 1)
            return carry

        lax.fori_loop(0, n_pairs, pair_body, 0)

        @pl.when(nc_w % 2 == 1)
        def _():
            drain(0, nc_w - 1)
            acc_store(0, nc_w - 1)

    return sc_kernel


def kernel(atomic_num, chiral_tag, degree, formal_charge, hybridization,
           is_aromatic, total_numHs, W_atomic_num, W_chiral_tag, W_degree,
           W_formal_charge, W_hybridization, W_is_aromatic, W_total_numHs):
    idxs = [atomic_num, chiral_tag, degree, formal_charge, hybridization,
            is_aromatic, total_numHs]
    tables = [W_atomic_num, W_chiral_tag, W_degree, W_formal_charge,
              W_hybridization, W_is_aromatic, W_total_numHs]
    n = atomic_num.shape[0]

    assert n % CHUNK == 0
    table = _build_merged_table(tables)
    grain = NW * L
    n_pad = ((n + grain - 1) // grain) * grain

    idx = jnp.stack([i.astype(jnp.int32) for i in idxs])
    idx = jnp.pad(idx, ((0, 0), (0, n_pad - n)))
    idx = idx.reshape(F, NW, n_pad // NW)

    return _make_sc_kernel(n, n_pad)(table, idx)


# speedup vs baseline: 1.0339x; 1.0072x over previous
import functools

import jax
import jax.numpy as jnp
from jax import lax
from jax.experimental import pallas as pl
from jax.experimental.pallas import tpu as pltpu
from jax.experimental.pallas import tpu_sc as plsc

D = 256
F = 7
NW = 32
CHUNK = 32
L = 16
NT = 3
BASES = (0, 868, 2623)


def _merge(wa, wc, wd, wf, wh, war, wn, o1, o2, o3):
    o1[...] = wa[...][:, None, :] + war[...][None, :, :]
    o2[...] = (
        wc[...][:, None, None, :]
        + wh[...][None, :, None, :]
        + wn[...][None, None, :, :]
    )
    o3[...] = wd[...][:, None, :] + wf[...][None, :, :]


def _build_merged_table(tables):
    wa, wc, wd, wf, wh, war, wn = tables
    o1, o2, o3 = pl.pallas_call(
        _merge,
        out_shape=[
            jax.ShapeDtypeStruct((124, 7, D), jnp.float32),
            jax.ShapeDtypeStruct((9, 13, 15, D), jnp.float32),
            jax.ShapeDtypeStruct((17, 22, D), jnp.float32),
        ],
    )(wa, wc, wd, wf, wh, war, wn)
    return jnp.concatenate(
        [o1.reshape(868, D), o2.reshape(1755, D), o3.reshape(374, D)], axis=0
    )


def _make_sc_kernel(n, n_pad):
    bpw = n_pad // NW
    mesh = plsc.VectorSubcoreMesh(core_axis_name="c", subcore_axis_name="s")

    @functools.partial(
        pl.kernel,
        mesh=mesh,
        out_type=jax.ShapeDtypeStruct((n, D), jnp.float32),
        scratch_types=(
            [pltpu.VMEM((bpw,), jnp.int32) for _ in range(F)]
            + [pltpu.VMEM((bpw,), jnp.int32) for _ in range(NT)]
            + [pltpu.VMEM((CHUNK, D), jnp.float32) for _ in range(2 * NT + 2)]
            + [pltpu.SemaphoreType.DMA for _ in range(4)]
        ),
    )
    def sc_kernel(table_hbm, idx_hbm, out_hbm, *scratch):
        raw = scratch[:F]
        cidx = scratch[F:F + NT]
        rows = (scratch[F + NT:F + 2 * NT], scratch[F + 2 * NT:F + 3 * NT])
        out_buf = scratch[F + 3 * NT:F + 3 * NT + 2]
        sems = scratch[F + 3 * NT + 2:F + 3 * NT + 4]
        out_sems = scratch[F + 3 * NT + 4:]
        wid = lax.axis_index("s") * 2 + lax.axis_index("c")
        base = wid * bpw
        nc_w = jnp.maximum(jnp.minimum(n - base, bpw), 0) // CHUNK
        n_pairs = nc_w // 2

        for f in range(F):
            pltpu.sync_copy(idx_hbm.at[f, wid], raw[f])

        def combine(g, carry):
            s = pl.ds(g * L, L)
            cidx[0][s] = raw[0][s] * 7 + raw[5][s]
            cidx[1][s] = (raw[1][s] * 13 + raw[4][s]) * 15 + raw[6][s] + BASES[1]
            cidx[2][s] = raw[2][s] * 22 + raw[3][s] + BASES[2]
            return carry

        lax.fori_loop(0, bpw // L, combine, 0)

        def issue(b, c):
            for t in range(NT):
                pltpu.async_copy(
                    table_hbm.at[cidx[t].at[pl.ds(c * CHUNK, CHUNK)]],
                    rows[b][t], sems[b],
                )

        def drain(b, c):
            for t in range(NT):
                pltpu.make_async_copy(
                    table_hbm.at[cidx[t].at[pl.ds(c * CHUNK, CHUNK)]],
                    rows[b][t], sems[b],
                ).wait()

        def acc_store(b, c):
            @pl.when(c >= 2)
            def _():
                pltpu.make_async_copy(
                    out_buf[b],
                    out_hbm.at[pl.ds(base + (c - 2) * CHUNK, CHUNK)],
                    out_sems[b],
                ).wait()

            def acc_row(r, carry2):
                for k in range(D // L):
                    s = pl.ds(k * L, L)
                    out_buf[b][r, s] = (
                        rows[b][0][r, s] + rows[b][1][r, s] + rows[b][2][r, s]
                    )
                return carry2

            lax.fori_loop(0, CHUNK, acc_row, 0)
            pltpu.async_copy(out_buf[b],
                             out_hbm.at[pl.ds(base + c * CHUNK, CHUNK)],
                             out_sems[b])

        issue(0, 0)

        def pair_body(i, carry):
            c0 = i * 2
            issue(1, c0 + 1)
            drain(0, c0)
            acc_store(0, c0)

            @pl.when(c0 + 2 < nc_w)
            def _():
                issue(0, c0 + 2)

            drain(1, c0 + 1)
            acc_store(1, c0 + 1)
            return carry

        lax.fori_loop(0, n_pairs, pair_body, 0)

        @pl.when(nc_w % 2 == 1)
        def _():
            drain(0, nc_w - 1)
            acc_store(0, nc_w - 1)

        for b in range(2):
            @pl.when(nc_w >= b + 1)
            def _(b=b):
                pltpu.make_async_copy(
                    out_buf[b], out_hbm.at[pl.ds(base, CHUNK)], out_sems[b]
                ).wait()

    return sc_kernel


def kernel(atomic_num, chiral_tag, degree, formal_charge, hybridization,
           is_aromatic, total_numHs, W_atomic_num, W_chiral_tag, W_degree,
           W_formal_charge, W_hybridization, W_is_aromatic, W_total_numHs):
    idxs = [atomic_num, chiral_tag, degree, formal_charge, hybridization,
            is_aromatic, total_numHs]
    tables = [W_atomic_num, W_chiral_tag, W_degree, W_formal_charge,
              W_hybridization, W_is_aromatic, W_total_numHs]
    n = atomic_num.shape[0]

    assert n % CHUNK == 0
    table = _build_merged_table(tables)
    grain = NW * L
    n_pad = ((n + grain - 1) // grain) * grain

    idx = jnp.stack([i.astype(jnp.int32) for i in idxs])
    idx = jnp.pad(idx, ((0, 0), (0, n_pad - n)))
    idx = idx.reshape(F, NW, n_pad // NW)

    return _make_sc_kernel(n, n_pad)(table, idx)


# speedup vs baseline: 1.0994x; 1.0633x over previous
import functools

import jax
import jax.numpy as jnp
from jax import lax
from jax.experimental import pallas as pl
from jax.experimental.pallas import tpu as pltpu
from jax.experimental.pallas import tpu_sc as plsc

D = 256
F = 7
NW = 32
CHUNK = 32
L = 16
NT = 3


def _merge(wa, wc, wd, wf, wh, war, wn, o1, o2, o3):
    o1[...] = wa[...][:, None, :] + war[...][None, :, :]
    o2[...] = (
        wc[...][:, None, None, :]
        + wh[...][None, :, None, :]
        + wn[...][None, None, :, :]
    )
    o3[...] = wd[...][:, None, :] + wf[...][None, :, :]


def _build_merged_table(tables):
    wa, wc, wd, wf, wh, war, wn = tables
    o1, o2, o3 = pl.pallas_call(
        _merge,
        out_shape=[
            jax.ShapeDtypeStruct((124, 7, D), jnp.float32),
            jax.ShapeDtypeStruct((9, 13, 15, D), jnp.float32),
            jax.ShapeDtypeStruct((17, 22, D), jnp.float32),
        ],
    )(wa, wc, wd, wf, wh, war, wn)
    return o1.reshape(868, D), o2.reshape(1755, D), o3.reshape(374, D)


def _make_sc_kernel(n, n_pad):
    bpw = n_pad // NW
    mesh = plsc.VectorSubcoreMesh(core_axis_name="c", subcore_axis_name="s")

    @functools.partial(
        pl.kernel,
        mesh=mesh,
        out_type=jax.ShapeDtypeStruct((n, D), jnp.float32),
        scratch_types=(
            [pltpu.VMEM((bpw,), jnp.int32) for _ in range(F)]
            + [pltpu.VMEM((bpw,), jnp.int32) for _ in range(NT)]
            + [pltpu.VMEM((CHUNK, D), jnp.float32) for _ in range(2 * NT + 2)]
            + [pltpu.SemaphoreType.DMA for _ in range(4)]
        ),
    )
    def sc_kernel(t1_hbm, t2_hbm, t3_hbm, idx_hbm, out_hbm, *scratch):
        tabs = (t1_hbm, t2_hbm, t3_hbm)
        raw = scratch[:F]
        cidx = scratch[F:F + NT]
        rows = (scratch[F + NT:F + 2 * NT], scratch[F + 2 * NT:F + 3 * NT])
        out_buf = scratch[F + 3 * NT:F + 3 * NT + 2]
        sems = scratch[F + 3 * NT + 2:F + 3 * NT + 4]
        out_sems = scratch[F + 3 * NT + 4:]
        wid = lax.axis_index("s") * 2 + lax.axis_index("c")
        base = wid * bpw
        nc_w = jnp.maximum(jnp.minimum(n - base, bpw), 0) // CHUNK
        n_pairs = nc_w // 2

        for f in range(F):
            pltpu.sync_copy(idx_hbm.at[f, wid], raw[f])

        def combine(g, carry):
            s = pl.ds(g * L, L)
            cidx[0][s] = raw[0][s] * 7 + raw[5][s]
            cidx[1][s] = (raw[1][s] * 13 + raw[4][s]) * 15 + raw[6][s]
            cidx[2][s] = raw[2][s] * 22 + raw[3][s]
            return carry

        lax.fori_loop(0, bpw // L, combine, 0)

        def issue(b, c):
            for t in range(NT):
                pltpu.async_copy(
                    tabs[t].at[cidx[t].at[pl.ds(c * CHUNK, CHUNK)]],
                    rows[b][t], sems[b],
                )

        def drain(b, c):
            for t in range(NT):
                pltpu.make_async_copy(
                    tabs[t].at[cidx[t].at[pl.ds(c * CHUNK, CHUNK)]],
                    rows[b][t], sems[b],
                ).wait()

        def acc_store(b, c):
            @pl.when(c >= 2)
            def _():
                pltpu.make_async_copy(
                    out_buf[b],
                    out_hbm.at[pl.ds(base + (c - 2) * CHUNK, CHUNK)],
                    out_sems[b],
                ).wait()

            def acc_row(r, carry2):
                for k in range(D // L):
                    s = pl.ds(k * L, L)
                    out_buf[b][r, s] = (
                        rows[b][0][r, s] + rows[b][1][r, s] + rows[b][2][r, s]
                    )
                return carry2

            lax.fori_loop(0, CHUNK, acc_row, 0)
            pltpu.async_copy(out_buf[b],
                             out_hbm.at[pl.ds(base + c * CHUNK, CHUNK)],
                             out_sems[b])

        issue(0, 0)

        def pair_body(i, carry):
            c0 = i * 2
            issue(1, c0 + 1)
            drain(0, c0)
            acc_store(0, c0)

            @pl.when(c0 + 2 < nc_w)
            def _():
                issue(0, c0 + 2)

            drain(1, c0 + 1)
            acc_store(1, c0 + 1)
            return carry

        lax.fori_loop(0, n_pairs, pair_body, 0)

        @pl.when(nc_w % 2 == 1)
        def _():
            drain(0, nc_w - 1)
            acc_store(0, nc_w - 1)

        for b in range(2):
            @pl.when(nc_w >= b + 1)
            def _(b=b):
                pltpu.make_async_copy(
                    out_buf[b], out_hbm.at[pl.ds(base, CHUNK)], out_sems[b]
                ).wait()

    return sc_kernel


def kernel(atomic_num, chiral_tag, degree, formal_charge, hybridization,
           is_aromatic, total_numHs, W_atomic_num, W_chiral_tag, W_degree,
           W_formal_charge, W_hybridization, W_is_aromatic, W_total_numHs):
    idxs = [atomic_num, chiral_tag, degree, formal_charge, hybridization,
            is_aromatic, total_numHs]
    tables = [W_atomic_num, W_chiral_tag, W_degree, W_formal_charge,
              W_hybridization, W_is_aromatic, W_total_numHs]
    n = atomic_num.shape[0]

    assert n % CHUNK == 0
    t1, t2, t3 = _build_merged_table(tables)
    grain = NW * L
    n_pad = ((n + grain - 1) // grain) * grain

    idx = jnp.stack([i.astype(jnp.int32) for i in idxs])
    idx = jnp.pad(idx, ((0, 0), (0, n_pad - n)))
    idx = idx.reshape(F, NW, n_pad // NW)

    return _make_sc_kernel(n, n_pad)(t1, t2, t3, idx)
